# baseline (speedup 1.0000x reference)
import functools

import jax
import jax.numpy as jnp
import numpy as np
from jax import lax
from jax.experimental import pallas as pl
from jax.experimental.pallas import tpu as pltpu
from jax.experimental.pallas import tpu_sc as plsc

_NC, _NS, _LANES = 2, 16, 16
_NW = _NC * _NS
_CHUNK = 640
_IDXG = 128


def _proj_body(x_ref, w1_ref, b1_ref, pu_ref, pv_ref):
    d = x_ref.shape[1]
    w1 = w1_ref[...]
    wu = w1[0:d] + w1[2 * d:3 * d]
    wv = w1[d:2 * d] - w1[2 * d:3 * d]
    xb = x_ref[...]
    pu_ref[...] = (
        jnp.dot(xb, wu, preferred_element_type=jnp.float32) + b1_ref[...]
    )
    pv_ref[...] = jnp.dot(xb, wv, preferred_element_type=jnp.float32)


def _sc_gather_body(e_half, u_hbm, v_hbm, pu_hbm, pv_hbm,
                    gu_hbm, gv_hbm,
                    u_v, v_v, iu_v, iv_v, gu_v, gv_v, sem):
    h = pu_hbm.shape[1]
    wid = lax.axis_index("s") * _NC + lax.axis_index("c")
    half = wid % 2
    wl = wid // 2
    n_chunks_half = e_half // _CHUNK
    n_floor = n_chunks_half // (_NW // 2)
    n_rem = n_chunks_half % (_NW // 2)
    n_w = n_floor + jnp.where(wl < n_rem, 1, 0)

    def chunk_body(c, _):
        row_off = (wl + c * (_NW // 2)) * _CHUNK
        src_off = half * e_half + row_off
        pltpu.sync_copy(u_hbm.at[pl.ds(src_off, _CHUNK)], u_v)
        pltpu.sync_copy(v_hbm.at[pl.ds(src_off, _CHUNK)], v_v)

        def lane_body(i, _):
            s = pl.ds(i * _LANES, _LANES)
            uu = u_v[s]
            vv = v_v[s]
            iu_v[s] = jnp.minimum(uu, vv)
            iv_v[s] = jnp.maximum(uu, vv)
            return 0

        lax.fori_loop(0, _CHUNK // _LANES, lane_body, 0)

        cps = []
        for j in range(_CHUNK // _IDXG):
            s = pl.ds(j * _IDXG, _IDXG)
            cps.append(pltpu.async_copy(pu_hbm.at[iu_v.at[s]], gu_v.at[s], sem))
            cps.append(pltpu.async_copy(pv_hbm.at[iv_v.at[s]], gv_v.at[s], sem))
        for cp in cps:
            cp.wait()

        lanes_iota = lax.iota(jnp.int32, _LANES)
        zeros = lanes_iota * 0

        def sign_body(i, _):
            s = pl.ds(i * _LANES, _LANES)
            sbit = jnp.where(u_v[s] > v_v[s], 1, 0)
            rows = lanes_iota + i * _LANES
            vals = plsc.load_gather(gu_v, [rows, zeros])
            ival = plsc.bitcast(vals, jnp.int32)
            ival = jnp.bitwise_or(jnp.bitwise_and(ival, -2), sbit)
            plsc.store_scatter(gu_v, [rows, zeros],
                               plsc.bitcast(ival, jnp.float32))
            return 0

        lax.fori_loop(0, _CHUNK // _LANES, sign_body, 0)

        dst_rows = pl.ds(row_off, _CHUNK)

        @pl.when(half == 0)
        def _():
            pltpu.sync_copy(gu_v, gu_hbm.at[dst_rows, pl.ds(0, h)])
            pltpu.sync_copy(gv_v, gv_hbm.at[dst_rows, pl.ds(0, h)])

        @pl.when(half == 1)
        def _():
            pltpu.sync_copy(gu_v, gu_hbm.at[dst_rows, pl.ds(h, h)])
            pltpu.sync_copy(gv_v, gv_hbm.at[dst_rows, pl.ds(h, h)])

        return 0

    lax.fori_loop(0, n_w, chunk_body, 0)


def _mlp_body(gu_ref, gv_ref, w2_ref, b2_ref, w3_ref, b3_ref,
              pt_ref, out_ref):
    h = w2_ref.shape[0]
    hh = pl.program_id(1)
    guv = gu_ref[...]
    gvv = gv_ref[...]
    gu_h = jnp.where(hh == 0, guv[:, 0:h], guv[:, h:2 * h])
    gv_h = jnp.where(hh == 0, gvv[:, 0:h], gvv[:, h:2 * h])
    bit = jnp.bitwise_and(
        lax.bitcast_convert_type(gu_h[:, 0:1], jnp.int32), 1)
    factor = (1 - 2 * bit).astype(jnp.float32)
    bf = jnp.bfloat16
    h1 = jnp.tanh(gu_h + gv_h)
    h2 = jnp.tanh(
        jnp.dot(h1.astype(bf), w2_ref[...].astype(bf),
                preferred_element_type=jnp.float32)
        + b2_ref[...])
    y = (jnp.dot(h2.astype(bf), w3_ref[...].astype(bf),
                 preferred_element_type=jnp.float32)
         + b3_ref[...])
    s = (1.5 * factor) * jnp.tanh(y)
    o = s - jnp.dot(s.astype(bf), pt_ref[...].astype(bf),
                    preferred_element_type=jnp.float32)
    out_ref[...] = o


def kernel(x, edges_uv, W1, b1, W2, b2, W3, b3):
    n, d = x.shape
    e = edges_uv.shape[0]
    h = W2.shape[0]
    kk = W3.shape[1]
    k = int(np.sqrt(kk))

    u = edges_uv[:, 0]
    v = edges_uv[:, 1]

    pu, pv = pl.pallas_call(
        _proj_body,
        out_shape=(
            jax.ShapeDtypeStruct((n, h), jnp.float32),
            jax.ShapeDtypeStruct((n, h), jnp.float32),
        ),
    )(x, W1, b1.reshape(1, h))

    mesh = plsc.VectorSubcoreMesh(
        core_axis_name="c", subcore_axis_name="s",
        num_cores=_NC, num_subcores=_NS)
    sc = pl.kernel(
        functools.partial(_sc_gather_body, e // 2),
        out_type=(
            jax.ShapeDtypeStruct((e // 2, 2 * h), jnp.float32),
            jax.ShapeDtypeStruct((e // 2, 2 * h), jnp.float32),
        ),
        mesh=mesh,
        scratch_types=(
            pltpu.VMEM((_CHUNK,), jnp.int32),
            pltpu.VMEM((_CHUNK,), jnp.int32),
            pltpu.VMEM((_CHUNK,), jnp.int32),
            pltpu.VMEM((_CHUNK,), jnp.int32),
            pltpu.VMEM((_CHUNK, h), jnp.float32),
            pltpu.VMEM((_CHUNK, h), jnp.float32),
            pltpu.SemaphoreType.DMA,
        ),
        compiler_params=pltpu.CompilerParams(
            use_tc_tiling_on_sc=False, needs_layout_passes=False),
    )
    gu2, gv2 = sc(u, v, pu, pv)

    ii = np.arange(kk)
    pt_np = np.zeros((kk, kk), dtype=np.float32)
    pt_np[(ii % k) * k + ii // k, ii] = 1.0
    pt = jnp.asarray(pt_np)

    be = 2000
    nb = e // 2 // be
    grid = (nb, 2)
    out = pl.pallas_call(
        _mlp_body,
        grid=grid,
        in_specs=[
            pl.BlockSpec((be, 2 * h), lambda i, hh: (i, 0)),
            pl.BlockSpec((be, 2 * h), lambda i, hh: (i, 0)),
            pl.BlockSpec((h, h), lambda i, hh: (0, 0)),
            pl.BlockSpec((1, h), lambda i, hh: (0, 0)),
            pl.BlockSpec((h, kk), lambda i, hh: (0, 0)),
            pl.BlockSpec((1, kk), lambda i, hh: (0, 0)),
            pl.BlockSpec((kk, kk), lambda i, hh: (0, 0)),
        ],
        out_specs=pl.BlockSpec((be, kk), lambda i, hh: (hh * nb + i, 0)),
        out_shape=jax.ShapeDtypeStruct((e, kk), jnp.float32),
    )(gu2, gv2, W2, b2.reshape(1, h), W3, b3.reshape(1, kk), pt)

    return out.reshape(e, k, k)


# speedup vs baseline: 1.0180x; 1.0180x over previous
import functools

import jax
import jax.numpy as jnp
import numpy as np
from jax import lax
from jax.experimental import pallas as pl
from jax.experimental.pallas import tpu as pltpu
from jax.experimental.pallas import tpu_sc as plsc

_NC, _NS, _LANES = 2, 16, 16
_NW = _NC * _NS
_CHUNK = 640
_IDXG = 128


def _proj_body(x_ref, w1_ref, b1_ref, pu_ref, pv_ref):
    d = x_ref.shape[1]
    w1 = w1_ref[...]
    wu = w1[0:d] + w1[2 * d:3 * d]
    wv = w1[d:2 * d] - w1[2 * d:3 * d]
    xb = x_ref[...]
    pu_ref[...] = (
        jnp.dot(xb, wu, preferred_element_type=jnp.float32) + b1_ref[...]
    )
    pv_ref[...] = jnp.dot(xb, wv, preferred_element_type=jnp.float32)


def _sc_gather_body(e_half, u_hbm, v_hbm, pu_hbm, pv_hbm,
                    gu_hbm, gv_hbm,
                    u_v, v_v, iu_v, iv_v, gu_v, gv_v, sem):
    h = pu_hbm.shape[1]
    wid = lax.axis_index("s") * _NC + lax.axis_index("c")
    half = wid % 2
    wl = wid // 2
    n_chunks_half = e_half // _CHUNK
    n_floor = n_chunks_half // (_NW // 2)
    n_rem = n_chunks_half % (_NW // 2)
    n_w = n_floor + jnp.where(wl < n_rem, 1, 0)

    def chunk_body(c, _):
        row_off = (wl + c * (_NW // 2)) * _CHUNK
        src_off = half * e_half + row_off
        pltpu.sync_copy(u_hbm.at[pl.ds(src_off, _CHUNK)], u_v)
        pltpu.sync_copy(v_hbm.at[pl.ds(src_off, _CHUNK)], v_v)

        def lane_body(i, _):
            s = pl.ds(i * _LANES, _LANES)
            uu = u_v[s]
            vv = v_v[s]
            iu_v[s] = jnp.minimum(uu, vv)
            iv_v[s] = jnp.maximum(uu, vv)
            return 0

        lax.fori_loop(0, _CHUNK // _LANES, lane_body, 0)

        cps = []
        for j in range(_CHUNK // _IDXG):
            s = pl.ds(j * _IDXG, _IDXG)
            cps.append(pltpu.async_copy(pu_hbm.at[iu_v.at[s]], gu_v.at[s], sem))
            cps.append(pltpu.async_copy(pv_hbm.at[iv_v.at[s]], gv_v.at[s], sem))
        for cp in cps:
            cp.wait()

        lanes_iota = lax.iota(jnp.int32, _LANES)
        zeros = lanes_iota * 0

        def sign_body(i, _):
            s = pl.ds(i * _LANES, _LANES)
            sbit = jnp.where(u_v[s] > v_v[s], 1, 0)
            rows = lanes_iota + i * _LANES
            vals = plsc.load_gather(gu_v, [rows, zeros])
            ival = plsc.bitcast(vals, jnp.int32)
            ival = jnp.bitwise_or(jnp.bitwise_and(ival, -2), sbit)
            plsc.store_scatter(gu_v, [rows, zeros],
                               plsc.bitcast(ival, jnp.float32))
            return 0

        lax.fori_loop(0, _CHUNK // _LANES, sign_body, 0)

        dst_rows = pl.ds(row_off, _CHUNK)

        @pl.when(half == 0)
        def _():
            pltpu.sync_copy(gu_v, gu_hbm.at[dst_rows, pl.ds(0, h)])
            pltpu.sync_copy(gv_v, gv_hbm.at[dst_rows, pl.ds(0, h)])

        @pl.when(half == 1)
        def _():
            pltpu.sync_copy(gu_v, gu_hbm.at[dst_rows, pl.ds(h, h)])
            pltpu.sync_copy(gv_v, gv_hbm.at[dst_rows, pl.ds(h, h)])

        return 0

    lax.fori_loop(0, n_w, chunk_body, 0)


def _mlp_body(gu_ref, gv_ref, w2_ref, b2_ref, w3_ref, b3_ref,
              pt_ref, out_ref):
    h = w2_ref.shape[0]
    hh = pl.program_id(1)
    guv = gu_ref[...]
    gvv = gv_ref[...]
    gu_h = jnp.where(hh == 0, guv[:, 0:h], guv[:, h:2 * h])
    gv_h = jnp.where(hh == 0, gvv[:, 0:h], gvv[:, h:2 * h])
    bit = jnp.bitwise_and(
        lax.bitcast_convert_type(gu_h[:, 0:1], jnp.int32), 1)
    factor = (1 - 2 * bit).astype(jnp.float32)
    h1 = jnp.tanh(gu_h + gv_h)
    h2 = jnp.tanh(
        jnp.dot(h1, w2_ref[...], preferred_element_type=jnp.float32)
        + b2_ref[...])
    y = (jnp.dot(h2, w3_ref[...], preferred_element_type=jnp.float32)
         + b3_ref[...])
    s = (1.5 * factor) * jnp.tanh(y)
    o = s - jnp.dot(s, pt_ref[...], preferred_element_type=jnp.float32)
    out_ref[...] = o


def kernel(x, edges_uv, W1, b1, W2, b2, W3, b3):
    n, d = x.shape
    e = edges_uv.shape[0]
    h = W2.shape[0]
    kk = W3.shape[1]
    k = int(np.sqrt(kk))

    u = edges_uv[:, 0]
    v = edges_uv[:, 1]

    pu, pv = pl.pallas_call(
        _proj_body,
        out_shape=(
            jax.ShapeDtypeStruct((n, h), jnp.float32),
            jax.ShapeDtypeStruct((n, h), jnp.float32),
        ),
    )(x, W1, b1.reshape(1, h))

    mesh = plsc.VectorSubcoreMesh(
        core_axis_name="c", subcore_axis_name="s",
        num_cores=_NC, num_subcores=_NS)
    sc = pl.kernel(
        functools.partial(_sc_gather_body, e // 2),
        out_type=(
            jax.ShapeDtypeStruct((e // 2, 2 * h), jnp.float32),
            jax.ShapeDtypeStruct((e // 2, 2 * h), jnp.float32),
        ),
        mesh=mesh,
        scratch_types=(
            pltpu.VMEM((_CHUNK,), jnp.int32),
            pltpu.VMEM((_CHUNK,), jnp.int32),
            pltpu.VMEM((_CHUNK,), jnp.int32),
            pltpu.VMEM((_CHUNK,), jnp.int32),
            pltpu.VMEM((_CHUNK, h), jnp.float32),
            pltpu.VMEM((_CHUNK, h), jnp.float32),
            pltpu.SemaphoreType.DMA,
        ),
        compiler_params=pltpu.CompilerParams(
            use_tc_tiling_on_sc=False, needs_layout_passes=False),
    )
    gu2, gv2 = sc(u, v, pu, pv)

    ii = np.arange(kk)
    pt_np = np.zeros((kk, kk), dtype=np.float32)
    pt_np[(ii % k) * k + ii // k, ii] = 1.0
    pt = jnp.asarray(pt_np)

    be = 2000
    nb = e // 2 // be
    grid = (nb, 2)
    out = pl.pallas_call(
        _mlp_body,
        grid=grid,
        in_specs=[
            pl.BlockSpec((be, 2 * h), lambda i, hh: (i, 0)),
            pl.BlockSpec((be, 2 * h), lambda i, hh: (i, 0)),
            pl.BlockSpec((h, h), lambda i, hh: (0, 0)),
            pl.BlockSpec((1, h), lambda i, hh: (0, 0)),
            pl.BlockSpec((h, kk), lambda i, hh: (0, 0)),
            pl.BlockSpec((1, kk), lambda i, hh: (0, 0)),
            pl.BlockSpec((kk, kk), lambda i, hh: (0, 0)),
        ],
        out_specs=pl.BlockSpec((be, kk), lambda i, hh: (hh * nb + i, 0)),
        out_shape=jax.ShapeDtypeStruct((e, kk), jnp.float32),
    )(gu2, gv2, W2, b2.reshape(1, h), W3, b3.reshape(1, kk), pt)

    return out.reshape(e, k, k)
